# baseline (speedup 1.0000x reference)
import functools

import jax
import jax.numpy as jnp
from jax import lax
from jax.experimental import pallas as pl
from jax.experimental.pallas import tpu as pltpu
from jax.experimental.pallas import tpu_sc as plsc

_VOCAB = 1000000
_DIM = 32
_B = 16384
_K = 20

_NC = 2
_NS = 16
_NW = _NC * _NS
_BPW = _B // _NW
_SUB = 16
_BSUB = _BPW // _SUB
_GRP = _BSUB // 16
_XROWS = _BSUB * _K
_XDMA = _XROWS // 128

_DBLK = 8192
_DROW = 262144
_JMASK = _DROW - 1


def _detile_body(a0, a1, a2, a3, b0, b1, b2, b3, oa, ob):
    eye = jnp.eye(128, dtype=jnp.float32)
    dn = (((0,), (0,)), ((), ()))
    sa = jnp.concatenate([a0[...], a1[...], a2[...], a3[...]], axis=0)
    sb = jnp.concatenate([b0[...], b1[...], b2[...], b3[...]], axis=0)
    oa[...] = lax.dot_general(sa, eye, dn, preferred_element_type=jnp.float32)
    ob[...] = lax.dot_general(sb, eye, dn, preferred_element_type=jnp.float32)


def _detile(ta, tb):
    last = _VOCAB // _DBLK
    mk = lambda a: pl.BlockSpec(
        (32, _DBLK),
        lambda i, A=a: (0, jnp.minimum(i + (_DROW // _DBLK) * A, last)))
    specs = [mk(a) for a in range(4)]
    return pl.pallas_call(
        _detile_body,
        grid=(_DROW // _DBLK,),
        in_specs=specs + specs,
        out_specs=[pl.BlockSpec((_DBLK, 128), lambda i: (i, 0))] * 2,
        out_shape=[jax.ShapeDtypeStruct((_DROW, 128), jnp.float32)] * 2,
        compiler_params=pltpu.CompilerParams(
            dimension_semantics=("parallel",)),
    )(ta, ta, ta, ta, tb, tb, tb, tb)


def _sc_body(cidx_hbm, xidx_hbm, din, dout,
             maxv_hbm, sume_hbm, ssum_hbm,
             cidx, xidx, crow, xrow,
             om, oe, os, sem0, sem1, sem2, sem3):
    wid = lax.axis_index("s") * _NC + lax.axis_index("c")
    iota = lax.iota(jnp.int32, 16)
    iota_k = iota * _K

    pltpu.sync_copy(cidx_hbm.at[pl.ds(wid * _BPW, _BPW)], cidx)
    pltpu.sync_copy(xidx_hbm.at[pl.ds(wid * _BPW * _K, _BPW * _K)], xidx)

    def remap_c(i, c):
        v = cidx[pl.ds(i * 16, 16)]
        cidx[pl.ds(i * 16, 16)] = ((v & _JMASK) << 2) | (v >> 18)
        return c
    lax.fori_loop(0, _BPW // 16, remap_c, 0)

    def remap_x(i, c):
        v = xidx[pl.ds(i * 16, 16)]
        xidx[pl.ds(i * 16, 16)] = ((v & _JMASK) << 2) | (v >> 18)
        return c
    lax.fori_loop(0, _BPW * _K // 16, remap_x, 0)

    sems = (sem0, sem1, sem2, sem3)

    def issue(s, buf):
        xbase = pl.multiple_of(s * _XROWS, 128)
        for i in range(_XDMA):
            pltpu.async_copy(dout.at[xidx.at[pl.ds(xbase + i * 128, 128)]],
                             xrow.at[pl.ds(buf * _XROWS + i * 128, 128)],
                             sems[buf])
        pltpu.async_copy(din.at[cidx.at[pl.ds(s * _BSUB, _BSUB)]],
                         crow.at[pl.ds(buf * _BSUB, _BSUB)], sems[buf])

    def drain(buf):
        pltpu.make_async_copy(dout.at[pl.ds(0, _XROWS)],
                              xrow.at[pl.ds(buf * _XROWS, _XROWS)],
                              sems[buf]).wait()
        pltpu.make_async_copy(din.at[pl.ds(0, _BSUB)],
                              crow.at[pl.ds(buf * _BSUB, _BSUB)],
                              sems[buf]).wait()

    def compute(s, buf):
        def grp_body(g, c2):
            bl0 = g * 16
            b0 = s * _BSUB + bl0
            slot_c = buf * _BSUB + bl0 + iota
            slot_k = [buf * _XROWS + bl0 * _K + k + iota_k
                      for k in range(_K)]
            acc = [jnp.zeros((16,), jnp.float32) for _ in range(_K)]
            for d in range(_DIM):
                dcol = jnp.full((16,), d, jnp.int32)
                cv = plsc.load_gather(crow, [slot_c, dcol])
                for k in range(_K):
                    xv = plsc.load_gather(xrow, [slot_k[k], dcol])
                    acc[k] = acc[k] + cv * xv
            m = acc[0]
            for k in range(1, _K):
                m = jnp.maximum(m, acc[k])
            e = jnp.exp(acc[0] - m)
            t = acc[0]
            for k in range(1, _K):
                e = e + jnp.exp(acc[k] - m)
                t = t + acc[k]
            om[pl.ds(b0, 16)] = m
            oe[pl.ds(b0, 16)] = e
            os[pl.ds(b0, 16)] = t
            return c2

        lax.fori_loop(0, _GRP, grp_body, 0)

    issue(0, 0)
    issue(1, 1)
    issue(2, 2)

    def ring_body(q, carry):
        for b in range(4):
            s = q * 4 + b
            drain(b)
            compute(s, b)
            issue(lax.rem(s + 3, _SUB), (b + 3) % 4)
        return carry

    lax.fori_loop(0, _SUB // 4, ring_body, 0)
    drain(0)
    drain(1)
    drain(2)

    pltpu.sync_copy(om, maxv_hbm.at[pl.ds(wid * _BPW, _BPW)])
    pltpu.sync_copy(oe, sume_hbm.at[pl.ds(wid * _BPW, _BPW)])
    pltpu.sync_copy(os, ssum_hbm.at[pl.ds(wid * _BPW, _BPW)])


_sc_score = functools.partial(
    pl.kernel,
    out_type=(jax.ShapeDtypeStruct((_B,), jnp.float32),) * 3,
    mesh=plsc.VectorSubcoreMesh(core_axis_name="c", subcore_axis_name="s",
                                num_cores=_NC, num_subcores=_NS),
    scratch_types=[
        pltpu.VMEM((_BPW,), jnp.int32),
        pltpu.VMEM((_BPW * _K,), jnp.int32),
        pltpu.VMEM((4 * _BSUB, 32), jnp.float32),
        pltpu.VMEM((4 * _XROWS, 32), jnp.float32),
        pltpu.VMEM((_BPW,), jnp.float32),
        pltpu.VMEM((_BPW,), jnp.float32),
        pltpu.VMEM((_BPW,), jnp.float32),
        pltpu.SemaphoreType.DMA,
        pltpu.SemaphoreType.DMA,
        pltpu.SemaphoreType.DMA,
        pltpu.SemaphoreType.DMA,
    ],
    compiler_params=pltpu.CompilerParams(needs_layout_passes=False,
                                         use_tc_tiling_on_sc=False),
)(_sc_body)


def _tc_finish_body(m_ref, e_ref, s_ref, o_ref):
    lse = m_ref[...] + jnp.log(e_ref[...])
    val = jnp.sum(lse) / _B - jnp.sum(s_ref[...]) / (_B * _K)
    o_ref[...] = jnp.reshape(val, (1, 1))


def kernel(center_ids, context_ids, in_table, out_table):
    cidx = center_ids.astype(jnp.int32)
    xidx = context_ids.astype(jnp.int32).reshape(_B * _K)
    din, dout = _detile(in_table.T, out_table.T)
    din = din.reshape(_DROW * 4, 32)
    dout = dout.reshape(_DROW * 4, 32)
    maxv, sume, ssum = _sc_score(cidx, xidx, din, dout)
    loss = pl.pallas_call(
        _tc_finish_body,
        out_shape=jax.ShapeDtypeStruct((1, 1), jnp.float32),
    )(maxv.reshape(128, 128), sume.reshape(128, 128), ssum.reshape(128, 128))
    return loss[0, 0]


# speedup vs baseline: 1.0074x; 1.0074x over previous
import functools

import jax
import jax.numpy as jnp
from jax import lax
from jax.experimental import pallas as pl
from jax.experimental.pallas import tpu as pltpu
from jax.experimental.pallas import tpu_sc as plsc

_VOCAB = 1000000
_DIM = 32
_B = 16384
_K = 20

_NC = 2
_NS = 16
_NW = _NC * _NS
_BPW = _B // _NW
_SUB = 8
_BSUB = _BPW // _SUB
_GRP = _BSUB // 16
_XROWS = _BSUB * _K
_XDMA = _XROWS // 128

_DBLK = 8192
_DROW = 262144
_JMASK = _DROW - 1


def _detile_body(a0, a1, a2, a3, b0, b1, b2, b3, oa, ob):
    eye = jnp.eye(128, dtype=jnp.float32)
    dn = (((0,), (0,)), ((), ()))
    sa = jnp.concatenate([a0[...], a1[...], a2[...], a3[...]], axis=0)
    sb = jnp.concatenate([b0[...], b1[...], b2[...], b3[...]], axis=0)
    oa[...] = lax.dot_general(sa, eye, dn, preferred_element_type=jnp.float32)
    ob[...] = lax.dot_general(sb, eye, dn, preferred_element_type=jnp.float32)


def _detile(ta, tb):
    last = _VOCAB // _DBLK
    mk = lambda a: pl.BlockSpec(
        (32, _DBLK),
        lambda i, A=a: (0, jnp.minimum(i + (_DROW // _DBLK) * A, last)))
    specs = [mk(a) for a in range(4)]
    return pl.pallas_call(
        _detile_body,
        grid=(_DROW // _DBLK,),
        in_specs=specs + specs,
        out_specs=[pl.BlockSpec((_DBLK, 128), lambda i: (i, 0))] * 2,
        out_shape=[jax.ShapeDtypeStruct((_DROW, 128), jnp.float32)] * 2,
        compiler_params=pltpu.CompilerParams(
            dimension_semantics=("parallel",)),
    )(ta, ta, ta, ta, tb, tb, tb, tb)


def _sc_body(cidx_hbm, xidx_hbm, din, dout,
             maxv_hbm, sume_hbm, ssum_hbm,
             cidx, xidx, crow, xrow,
             om, oe, os, sem0, sem1):
    wid = lax.axis_index("s") * _NC + lax.axis_index("c")
    iota = lax.iota(jnp.int32, 16)
    iota_k = iota * _K

    pltpu.sync_copy(cidx_hbm.at[pl.ds(wid * _BPW, _BPW)], cidx)
    pltpu.sync_copy(xidx_hbm.at[pl.ds(wid * _BPW * _K, _BPW * _K)], xidx)

    def remap_c(i, c):
        v = cidx[pl.ds(i * 16, 16)]
        cidx[pl.ds(i * 16, 16)] = ((v & _JMASK) << 2) | (v >> 18)
        return c
    lax.fori_loop(0, _BPW // 16, remap_c, 0)

    def remap_x(i, c):
        v = xidx[pl.ds(i * 16, 16)]
        xidx[pl.ds(i * 16, 16)] = ((v & _JMASK) << 2) | (v >> 18)
        return c
    lax.fori_loop(0, _BPW * _K // 16, remap_x, 0)

    sems = (sem0, sem1)

    def issue(s, buf):
        xbase = pl.multiple_of(s * _XROWS, 128)
        for i in range(_XDMA):
            pltpu.async_copy(dout.at[xidx.at[pl.ds(xbase + i * 128, 128)]],
                             xrow.at[pl.ds(buf * _XROWS + i * 128, 128)],
                             sems[buf])
        pltpu.async_copy(din.at[cidx.at[pl.ds(s * _BSUB, _BSUB)]],
                         crow.at[pl.ds(buf * _BSUB, _BSUB)], sems[buf])

    def drain(buf):
        pltpu.make_async_copy(dout.at[pl.ds(0, _XROWS)],
                              xrow.at[pl.ds(buf * _XROWS, _XROWS)],
                              sems[buf]).wait()
        pltpu.make_async_copy(din.at[pl.ds(0, _BSUB)],
                              crow.at[pl.ds(buf * _BSUB, _BSUB)],
                              sems[buf]).wait()

    def compute(s, buf):
        def grp_body(g, c2):
            bl0 = g * 16
            b0 = s * _BSUB + bl0
            slot_c = buf * _BSUB + bl0 + iota
            slot_k = [buf * _XROWS + bl0 * _K + k + iota_k
                      for k in range(_K)]
            acc = [jnp.zeros((16,), jnp.float32) for _ in range(_K)]
            for d in range(_DIM):
                dcol = jnp.full((16,), d, jnp.int32)
                cv = plsc.load_gather(crow, [slot_c, dcol])
                for k in range(_K):
                    xv = plsc.load_gather(xrow, [slot_k[k], dcol])
                    acc[k] = acc[k] + cv * xv
            m = acc[0]
            for k in range(1, _K):
                m = jnp.maximum(m, acc[k])
            e = jnp.exp(acc[0] - m)
            t = acc[0]
            for k in range(1, _K):
                e = e + jnp.exp(acc[k] - m)
                t = t + acc[k]
            om[pl.ds(b0, 16)] = m
            oe[pl.ds(b0, 16)] = e
            os[pl.ds(b0, 16)] = t
            return c2

        lax.fori_loop(0, _GRP, grp_body, 0)

    issue(0, 0)

    def pair_body(p, carry):
        s0 = p * 2
        issue(s0 + 1, 1)
        drain(0)
        compute(s0, 0)
        issue(lax.rem(s0 + 2, _SUB), 0)
        drain(1)
        compute(s0 + 1, 1)
        return carry

    lax.fori_loop(0, _SUB // 2, pair_body, 0)
    drain(0)

    pltpu.sync_copy(om, maxv_hbm.at[pl.ds(wid * _BPW, _BPW)])
    pltpu.sync_copy(oe, sume_hbm.at[pl.ds(wid * _BPW, _BPW)])
    pltpu.sync_copy(os, ssum_hbm.at[pl.ds(wid * _BPW, _BPW)])


_sc_score = functools.partial(
    pl.kernel,
    out_type=(jax.ShapeDtypeStruct((_B,), jnp.float32),) * 3,
    mesh=plsc.VectorSubcoreMesh(core_axis_name="c", subcore_axis_name="s",
                                num_cores=_NC, num_subcores=_NS),
    scratch_types=[
        pltpu.VMEM((_BPW,), jnp.int32),
        pltpu.VMEM((_BPW * _K,), jnp.int32),
        pltpu.VMEM((2 * _BSUB, 32), jnp.float32),
        pltpu.VMEM((2 * _XROWS, 32), jnp.float32),
        pltpu.VMEM((_BPW,), jnp.float32),
        pltpu.VMEM((_BPW,), jnp.float32),
        pltpu.VMEM((_BPW,), jnp.float32),
        pltpu.SemaphoreType.DMA,
        pltpu.SemaphoreType.DMA,
    ],
    compiler_params=pltpu.CompilerParams(needs_layout_passes=False,
                                         use_tc_tiling_on_sc=False),
)(_sc_body)


def _tc_finish_body(m_ref, e_ref, s_ref, o_ref):
    lse = m_ref[...] + jnp.log(e_ref[...])
    val = jnp.sum(lse) / _B - jnp.sum(s_ref[...]) / (_B * _K)
    o_ref[...] = jnp.reshape(val, (1, 1))


def kernel(center_ids, context_ids, in_table, out_table):
    cidx = center_ids.astype(jnp.int32)
    xidx = context_ids.astype(jnp.int32).reshape(_B * _K)
    din, dout = _detile(in_table.T, out_table.T)
    din = din.reshape(_DROW * 4, 32)
    dout = dout.reshape(_DROW * 4, 32)
    maxv, sume, ssum = _sc_score(cidx, xidx, din, dout)
    loss = pl.pallas_call(
        _tc_finish_body,
        out_shape=jax.ShapeDtypeStruct((1, 1), jnp.float32),
    )(maxv.reshape(128, 128), sume.reshape(128, 128), ssum.reshape(128, 128))
    return loss[0, 0]


# speedup vs baseline: 1.0114x; 1.0039x over previous
import functools

import jax
import jax.numpy as jnp
from jax import lax
from jax.experimental import pallas as pl
from jax.experimental.pallas import tpu as pltpu
from jax.experimental.pallas import tpu_sc as plsc

_VOCAB = 1000000
_DIM = 32
_B = 16384
_K = 20

_NC = 2
_NS = 16
_NW = _NC * _NS
_BPW = _B // _NW
_SUB = 16
_BSUB = _BPW // _SUB
_GRP = _BSUB // 16
_XROWS = _BSUB * _K
_XDMA = _XROWS // 128

_DBLK = 8192
_DROW = 262144
_JMASK = _DROW - 1


def _detile_body(a0, a1, a2, a3, b0, b1, b2, b3, oa, ob):
    eye = jnp.eye(128, dtype=jnp.float32)
    dn = (((0,), (0,)), ((), ()))
    sa = jnp.concatenate([a0[...], a1[...], a2[...], a3[...]], axis=0)
    sb = jnp.concatenate([b0[...], b1[...], b2[...], b3[...]], axis=0)
    oa[...] = lax.dot_general(sa, eye, dn, preferred_element_type=jnp.float32)
    ob[...] = lax.dot_general(sb, eye, dn, preferred_element_type=jnp.float32)


def _detile(ta, tb):
    last = _VOCAB // _DBLK
    mk = lambda a: pl.BlockSpec(
        (32, _DBLK),
        lambda i, A=a: (0, jnp.minimum(i + (_DROW // _DBLK) * A, last)))
    specs = [mk(a) for a in range(4)]
    return pl.pallas_call(
        _detile_body,
        grid=(_DROW // _DBLK,),
        in_specs=specs + specs,
        out_specs=[pl.BlockSpec((_DBLK, 128), lambda i: (i, 0))] * 2,
        out_shape=[jax.ShapeDtypeStruct((_DROW, 128), jnp.float32)] * 2,
        compiler_params=pltpu.CompilerParams(
            dimension_semantics=("parallel",)),
    )(ta, ta, ta, ta, tb, tb, tb, tb)


def _sc_body(cidx_hbm, xidx_hbm, din, dout,
             maxv_hbm, sume_hbm, ssum_hbm,
             cidx, xidx, crow, xrow,
             om, oe, os, sem0, sem1):
    wid = lax.axis_index("s") * _NC + lax.axis_index("c")
    iota = lax.iota(jnp.int32, 16)
    iota_k = iota * _K

    pltpu.sync_copy(cidx_hbm.at[pl.ds(wid * _BPW, _BPW)], cidx)
    pltpu.sync_copy(xidx_hbm.at[pl.ds(wid * _BPW * _K, _BPW * _K)], xidx)

    def remap_c(i, c):
        v = cidx[pl.ds(i * 16, 16)]
        cidx[pl.ds(i * 16, 16)] = ((v & _JMASK) << 2) | (v >> 18)
        return c
    lax.fori_loop(0, _BPW // 16, remap_c, 0)

    def remap_x(i, c):
        v = xidx[pl.ds(i * 16, 16)]
        xidx[pl.ds(i * 16, 16)] = ((v & _JMASK) << 2) | (v >> 18)
        return c
    lax.fori_loop(0, _BPW * _K // 16, remap_x, 0)

    sems = (sem0, sem1)

    def issue(s, buf):
        xbase = pl.multiple_of(s * _XROWS, 128)
        for i in range(_XDMA):
            pltpu.async_copy(dout.at[xidx.at[pl.ds(xbase + i * 128, 128)]],
                             xrow.at[pl.ds(buf * _XROWS + i * 128, 128)],
                             sems[buf])
        pltpu.async_copy(din.at[cidx.at[pl.ds(s * _BSUB, _BSUB)]],
                         crow.at[pl.ds(buf * _BSUB, _BSUB)], sems[buf])

    def drain(buf):
        pltpu.make_async_copy(dout.at[pl.ds(0, _XROWS)],
                              xrow.at[pl.ds(buf * _XROWS, _XROWS)],
                              sems[buf]).wait()
        pltpu.make_async_copy(din.at[pl.ds(0, _BSUB)],
                              crow.at[pl.ds(buf * _BSUB, _BSUB)],
                              sems[buf]).wait()

    def compute(s, buf):
        def grp_body(g, c2):
            bl0 = g * 16
            b0 = s * _BSUB + bl0
            slot_c = buf * _BSUB + bl0 + iota
            slot_k = [buf * _XROWS + bl0 * _K + k + iota_k
                      for k in range(_K)]
            acc = [jnp.zeros((16,), jnp.float32) for _ in range(_K)]
            for d in range(_DIM):
                dcol = jnp.full((16,), d, jnp.int32)
                cv = plsc.load_gather(crow, [slot_c, dcol])
                for k in range(_K):
                    xv = plsc.load_gather(xrow, [slot_k[k], dcol])
                    acc[k] = acc[k] + cv * xv
            m = acc[0]
            for k in range(1, _K):
                m = jnp.maximum(m, acc[k])
            e = jnp.exp(acc[0] - m)
            t = acc[0]
            for k in range(1, _K):
                e = e + jnp.exp(acc[k] - m)
                t = t + acc[k]
            om[pl.ds(b0, 16)] = m
            oe[pl.ds(b0, 16)] = e
            os[pl.ds(b0, 16)] = t
            return c2

        lax.fori_loop(0, _GRP, grp_body, 0)

    issue(0, 0)

    def pair_body(p, carry):
        s0 = p * 2
        issue(s0 + 1, 1)
        drain(0)
        compute(s0, 0)
        issue(lax.rem(s0 + 2, _SUB), 0)
        drain(1)
        compute(s0 + 1, 1)
        return carry

    lax.fori_loop(0, _SUB // 2, pair_body, 0)
    drain(0)

    pltpu.sync_copy(om, maxv_hbm.at[pl.ds(wid * _BPW, _BPW)])
    pltpu.sync_copy(oe, sume_hbm.at[pl.ds(wid * _BPW, _BPW)])
    pltpu.sync_copy(os, ssum_hbm.at[pl.ds(wid * _BPW, _BPW)])


_sc_score = functools.partial(
    pl.kernel,
    out_type=(jax.ShapeDtypeStruct((_B,), jnp.float32),) * 3,
    mesh=plsc.VectorSubcoreMesh(core_axis_name="c", subcore_axis_name="s",
                                num_cores=_NC, num_subcores=_NS),
    scratch_types=[
        pltpu.VMEM((_BPW,), jnp.int32),
        pltpu.VMEM((_BPW * _K,), jnp.int32),
        pltpu.VMEM((2 * _BSUB, 32), jnp.float32),
        pltpu.VMEM((2 * _XROWS, 32), jnp.float32),
        pltpu.VMEM((_BPW,), jnp.float32),
        pltpu.VMEM((_BPW,), jnp.float32),
        pltpu.VMEM((_BPW,), jnp.float32),
        pltpu.SemaphoreType.DMA,
        pltpu.SemaphoreType.DMA,
    ],
    compiler_params=pltpu.CompilerParams(needs_layout_passes=False,
                                         use_tc_tiling_on_sc=False),
)(_sc_body)


def _tc_finish_body(m_ref, e_ref, s_ref, o_ref):
    lse = m_ref[...] + jnp.log(e_ref[...])
    val = jnp.sum(lse) / _B - jnp.sum(s_ref[...]) / (_B * _K)
    o_ref[...] = jnp.reshape(val, (1, 1))


def kernel(center_ids, context_ids, in_table, out_table):
    cidx = center_ids.astype(jnp.int32)
    xidx = context_ids.astype(jnp.int32).reshape(_B * _K)
    din, dout = _detile(in_table.T, out_table.T)
    din = din.reshape(_DROW * 4, 32)
    dout = dout.reshape(_DROW * 4, 32)
    maxv, sume, ssum = _sc_score(cidx, xidx, din, dout)
    loss = pl.pallas_call(
        _tc_finish_body,
        out_shape=jax.ShapeDtypeStruct((1, 1), jnp.float32),
    )(maxv.reshape(128, 128), sume.reshape(128, 128), ssum.reshape(128, 128))
    return loss[0, 0]
